# baseline (speedup 1.0000x reference)
import functools

import jax
import jax.numpy as jnp
from jax import lax
from jax.experimental import pallas as pl
from jax.experimental.pallas import tpu as pltpu, tpu_sc as plsc

_B = 16384
_TABLE = 1000
_TABLE_PAD = 1024
_L = 16


def _make_gather():
    info = plsc.get_sparse_core_info()
    nc, ns = 1, info.num_subcores
    nw = nc * ns
    b_per_w = _B // nw
    mesh = plsc.VectorSubcoreMesh(
        core_axis_name="c", subcore_axis_name="s", num_cores=nc
    )

    @functools.partial(
        pl.kernel,
        mesh=mesh,
        compiler_params=pltpu.CompilerParams(
            needs_layout_passes=False,
            skip_device_barrier=True,
            disable_bounds_checks=True,
            disable_semaphore_checks=True,
        ),
        out_type=jax.ShapeDtypeStruct((_B,), jnp.float32),
        scratch_types=[
            pltpu.VMEM((_TABLE_PAD,), jnp.float32),
            pltpu.VMEM((b_per_w,), jnp.int32),
            pltpu.VMEM((b_per_w,), jnp.float32),
            pltpu.SemaphoreType.DMA,
            pltpu.SemaphoreType.DMA,
            pltpu.SemaphoreType.DMA,
        ],
    )
    def gather_k(
        kp_hbm, step_hbm, out_hbm, table_v, idx_v, res_v, sem_t, sem_i, sem_o
    ):
        wid = lax.axis_index("s") * nc + lax.axis_index("c")
        base = wid * b_per_w
        c_t = pltpu.async_copy(kp_hbm, table_v.at[pl.ds(0, _TABLE)], sem_t)
        c_i = pltpu.async_copy(step_hbm.at[pl.ds(base, b_per_w)], idx_v, sem_i)
        c_i.wait()
        c_t.wait()
        n_chunk = 4
        ch = b_per_w // n_chunk
        stores = []
        for c in range(n_chunk):
            @pl.loop(c * ch, (c + 1) * ch, step=_L, unroll=8)
            def _(off):
                idx = idx_v[pl.ds(off, _L)]
                res_v[pl.ds(off, _L)] = plsc.load_gather(table_v, [idx])
            stores.append(
                pltpu.async_copy(
                    res_v.at[pl.ds(c * ch, ch)],
                    out_hbm.at[pl.ds(base + c * ch, ch)],
                    sem_o,
                )
            )
        for s in stores:
            s.wait()

    return gather_k


def kernel(kp, step):
    return _make_gather()(kp, step.astype(jnp.int32))


# speedup vs baseline: 1.0107x; 1.0107x over previous
import functools

import jax
import jax.numpy as jnp
from jax import lax
from jax.experimental import pallas as pl
from jax.experimental.pallas import tpu as pltpu, tpu_sc as plsc

_B = 16384
_TABLE = 1000
_TABLE_PAD = 1024
_L = 16


def _make_gather():
    info = plsc.get_sparse_core_info()
    nc, ns = 1, info.num_subcores
    nw = nc * ns
    b_per_w = _B // nw
    mesh = plsc.VectorSubcoreMesh(
        core_axis_name="c", subcore_axis_name="s", num_cores=nc
    )

    @functools.partial(
        pl.kernel,
        mesh=mesh,
        compiler_params=pltpu.CompilerParams(
            needs_layout_passes=False,
            skip_device_barrier=True,
            disable_bounds_checks=True,
            disable_semaphore_checks=True,
        ),
        out_type=jax.ShapeDtypeStruct((_B,), jnp.float32),
        scratch_types=[
            pltpu.VMEM((_TABLE_PAD,), jnp.float32),
            pltpu.VMEM((b_per_w,), jnp.int32),
            pltpu.VMEM((b_per_w,), jnp.float32),
            pltpu.SemaphoreType.DMA,
            pltpu.SemaphoreType.DMA,
        ],
    )
    def gather_k(kp_hbm, step_hbm, out_hbm, table_v, idx_v, res_v, sem_t, sem_i):
        wid = lax.axis_index("s") * nc + lax.axis_index("c")
        base = wid * b_per_w
        c_t = pltpu.async_copy(kp_hbm, table_v.at[pl.ds(0, _TABLE)], sem_t)
        c_i = pltpu.async_copy(step_hbm.at[pl.ds(base, b_per_w)], idx_v, sem_i)
        c_i.wait()
        c_t.wait()
        @pl.loop(0, b_per_w, step=_L, unroll=8)
        def _(off):
            idx = idx_v[pl.ds(off, _L)]
            res_v[pl.ds(off, _L)] = plsc.load_gather(table_v, [idx])
        pltpu.sync_copy(res_v, out_hbm.at[pl.ds(base, b_per_w)])

    return gather_k


def kernel(kp, step):
    return _make_gather()(kp, step.astype(jnp.int32))


# speedup vs baseline: 1.0281x; 1.0172x over previous
import functools

import jax
import jax.numpy as jnp
from jax import lax
from jax.experimental import pallas as pl
from jax.experimental.pallas import tpu as pltpu, tpu_sc as plsc

_B = 16384
_TABLE = 1000
_TABLE_PAD = 1024
_L = 16


def _make_gather():
    info = plsc.get_sparse_core_info()
    nc, ns = 1, info.num_subcores
    nw = nc * ns
    b_per_w = _B // nw
    mesh = plsc.VectorSubcoreMesh(
        core_axis_name="c", subcore_axis_name="s", num_cores=nc
    )

    @functools.partial(
        pl.kernel,
        mesh=mesh,
        compiler_params=pltpu.CompilerParams(
            needs_layout_passes=False,
            skip_device_barrier=True,
            disable_bounds_checks=True,
            disable_semaphore_checks=True,
        ),
        out_type=jax.ShapeDtypeStruct((_B,), jnp.float32),
        scratch_types=[
            pltpu.VMEM((_TABLE_PAD,), jnp.float32),
            pltpu.VMEM((b_per_w,), jnp.int32),
            pltpu.VMEM((b_per_w,), jnp.float32),
            pltpu.SemaphoreType.DMA,
            pltpu.SemaphoreType.DMA,
        ],
    )
    def gather_k(kp_hbm, step_hbm, out_hbm, table_v, idx_v, res_v, sem_t, sem_i):
        wid = lax.axis_index("s") * nc + lax.axis_index("c")
        base = wid * b_per_w
        c_t = pltpu.async_copy(kp_hbm, table_v.at[pl.ds(0, _TABLE)], sem_t)
        c_i = pltpu.async_copy(step_hbm.at[pl.ds(base, b_per_w)], idx_v, sem_i)
        c_i.wait()
        c_t.wait()
        @plsc.parallel_loop(0, b_per_w, _L, unroll=8)
        def _(off):
            idx = idx_v[pl.ds(off, _L)]
            res_v[pl.ds(off, _L)] = plsc.load_gather(table_v, [idx])
        pltpu.sync_copy(res_v, out_hbm.at[pl.ds(base, b_per_w)])

    return gather_k


def kernel(kp, step):
    return _make_gather()(kp, step.astype(jnp.int32))
